# baseline (speedup 1.0000x reference)
import functools

import jax
import jax.numpy as jnp
import numpy as np
from jax import lax
from jax.experimental import pallas as pl
from jax.experimental.pallas import tpu as pltpu
from jax.experimental.pallas import tpu_sc as plsc

_NB, _NCH, _NH, _NW = 32, 15, 128, 128
_NA = _NCH // 5
_NPLANES = _NB * _NCH
_GS = _NH
_TC_BLK = 160


def _tc_body(x_ref, o_ref):
    inv_gs = np.float32(1.0 / _GS)
    gx = lax.broadcasted_iota(jnp.int32, (_NH, _NW), 1).astype(jnp.float32) * inv_gs
    gy = lax.broadcasted_iota(jnp.int32, (_NH, _NW), 0).astype(jnp.float32) * inv_gs
    for c in range(_TC_BLK):
        s = jnp.tanh(x_ref[c] * np.float32(0.5)) * np.float32(0.5) + np.float32(0.5)
        if c % 5 == 0:
            s = s * inv_gs + gx
        elif c % 5 == 1:
            s = s * inv_gs + gy
        o_ref[c] = s


def _tc_decode(x, n_planes):
    return pl.pallas_call(
        _tc_body,
        grid=(n_planes // _TC_BLK,),
        in_specs=[pl.BlockSpec((_TC_BLK, _NH, _NW), lambda i: (i, 0, 0))],
        out_specs=pl.BlockSpec((_TC_BLK, _NH, _NW), lambda i: (i, 0, 0)),
        out_shape=jax.ShapeDtypeStruct((n_planes, _NH, _NW), jnp.float32),
    )(x)


def kernel(out, infer):
    del infer
    x = out.reshape(_NPLANES, _NH, _NW)
    y = _tc_decode(x, _NPLANES)
    return jnp.transpose(y.reshape(_NB, _NA, 5, _NH, _NW), (0, 1, 3, 4, 2))


_NWORKERS = 32
_SC_PL_PER_W = _NPLANES // _NWORKERS


def _sc_body(in_hbm, out_hbm, ib0, ib1, ob0, ob1, si0, si1, so0, so1):
    nc = 2
    wid = lax.axis_index("s") * nc + lax.axis_index("c")
    base = wid * _SC_PL_PER_W
    iotaf = lax.iota(jnp.int32, 16).astype(jnp.float32)
    inv_gs = np.float32(1.0 / _GS)
    ibufs, obufs = (ib0, ib1), (ob0, ob1)
    isems, osems = (si0, si1), (so0, so1)

    def compute(c, ib, ob):
        def row(v, carry):
            vf = v.astype(jnp.float32) * inv_gs
            for u in range(_NW // 16):
                x = ib[v, pl.ds(u * 16, 16)]
                s = 1.0 / (1.0 + jnp.exp(-x))
                if c == 0:
                    s = s * inv_gs + (iotaf + np.float32(16 * u)) * inv_gs
                elif c == 1:
                    s = s * inv_gs + vf
                ob[v, pl.ds(u * 16, 16)] = s
            return carry

        lax.fori_loop(0, _NH, row, 0)

    in_handles = [None, None]
    out_handles = [None, None]
    in_handles[0] = pltpu.async_copy(in_hbm.at[base], ibufs[0], isems[0])
    for k in range(_SC_PL_PER_W):
        b = k % 2
        if k + 1 < _SC_PL_PER_W:
            in_handles[1 - b] = pltpu.async_copy(
                in_hbm.at[base + (k + 1)], ibufs[1 - b], isems[1 - b])
        in_handles[b].wait()
        if out_handles[b] is not None:
            out_handles[b].wait()
        compute(k % 5, ibufs[b], obufs[b])
        out_handles[b] = pltpu.async_copy(obufs[b], out_hbm.at[base + k],
                                          osems[b])
    for h in out_handles:
        if h is not None:
            h.wait()


def _sc_decode(x):
    mesh = plsc.VectorSubcoreMesh(core_axis_name="c", subcore_axis_name="s")
    run = functools.partial(
        pl.kernel,
        mesh=mesh,
        compiler_params=pltpu.CompilerParams(needs_layout_passes=False),
        out_type=jax.ShapeDtypeStruct((_NPLANES, _NH, _NW), jnp.float32),
        scratch_types=[
            pltpu.VMEM((_NH, _NW), jnp.float32),
            pltpu.VMEM((_NH, _NW), jnp.float32),
            pltpu.VMEM((_NH, _NW), jnp.float32),
            pltpu.VMEM((_NH, _NW), jnp.float32),
            pltpu.SemaphoreType.DMA,
            pltpu.SemaphoreType.DMA,
            pltpu.SemaphoreType.DMA,
            pltpu.SemaphoreType.DMA,
        ],
    )(_sc_body)
    return run(x)
